# baseline (speedup 1.0000x reference)
import functools

import jax
import jax.numpy as jnp
import numpy as np
from jax import lax
from jax.experimental import pallas as pl
from jax.experimental.pallas import tpu as pltpu
from jax.experimental.pallas import tpu_sc as plsc

NWORK = 32
NBINS = 65536
SENTINEL = np.uint32(0xFFFFFFFF)
_MESH = dict(core_axis_name="c", subcore_axis_name="s", num_cores=2)


def _keys_from_f32(x):
    u = plsc.bitcast(x, jnp.uint32)
    u = jnp.where(u == jnp.uint32(0x80000000), jnp.uint32(0), u)
    neg = u >= jnp.uint32(0x80000000)
    return jnp.where(neg, u, (u ^ SENTINEL) & jnp.uint32(0x7FFFFFFF))


def _zero_hist(hist):
    def z(i, _):
        hist[pl.ds(i * 16, 16)] = jnp.zeros((16,), jnp.int32)
        return 0
    lax.fori_loop(0, NBINS // 16, z, 0)


def _scan_hist(hist, bias):
    def s(i, c):
        v = hist[pl.ds(i * 16, 16)]
        hist[pl.ds(i * 16, 16)] = c + plsc.cumsum(v) - v
        return c + jnp.sum(v)
    lax.fori_loop(0, NBINS // 16, s, bias)


def _scatter_chunks(posb, sem, nchunk, pairs):
    for src_ref, dst_ref in pairs:
        for c in range(nchunk):
            pltpu.async_copy(
                src_ref.at[pl.ds(c * 128, 128)],
                dst_ref.at[posb.at[c]],
                sem,
            ).wait()


def _build_pass1(rows, n, w):
    npad = -(-n // w) * w
    nwin = npad // w
    vpw = w // 16
    nfull = n // w
    tail = n - nfull * w
    nchunk = w // 128
    rpw = rows // NWORK
    assert rows % NWORK == 0 and w % 128 == 0 and tail % 8 == 0

    def body(x_hbm, keys2, vals2, win_f, posb, tkb, tvb, hist, sem):
        wid = lax.axis_index("s") * 2 + lax.axis_index("c")
        lanes = lax.iota(jnp.int32, 16)
        ones = jnp.ones((16,), jnp.int32)

        def row_loop(j, _):
            r = wid * rpw + j
            rbase = r * npad

            def load_x(w_):
                @pl.when(w_ < nfull)
                def _():
                    pltpu.sync_copy(x_hbm.at[pl.ds(r * n + w_ * w, w)], win_f)
                if tail:
                    @pl.when(w_ == nfull)
                    def _():
                        pltpu.sync_copy(
                            x_hbm.at[pl.ds(r * n + nfull * w, tail)],
                            win_f.at[pl.ds(0, tail)])

            _zero_hist(hist)

            def h_win(w_, _):
                load_x(w_)

                def h_v(i, _):
                    gbase = w_ * w + i * 16
                    valid = (gbase + lanes) < n
                    k = _keys_from_f32(win_f[pl.ds(i * 16, 16)])
                    k = jnp.where(valid, k, SENTINEL)
                    d = (k & jnp.uint32(0xFFFF)).astype(jnp.int32)
                    plsc.addupdate_scatter(hist, [d], ones)
                    return 0
                lax.fori_loop(0, vpw, h_v, 0)
                return 0
            lax.fori_loop(0, nwin, h_win, 0)

            _scan_hist(hist, rbase)

            def p_win(w_, _):
                load_x(w_)

                def p_v(i, _):
                    gbase = w_ * w + i * 16
                    valid = (gbase + lanes) < n
                    k = _keys_from_f32(win_f[pl.ds(i * 16, 16)])
                    k = jnp.where(valid, k, SENTINEL)
                    d = (k & jnp.uint32(0xFFFF)).astype(jnp.int32)
                    cnt, _last = plsc.scan_count(d)
                    base = plsc.load_gather(hist, [d])
                    pos = base + cnt - 1
                    plsc.addupdate_scatter(hist, [d], ones)
                    tkb[pl.ds(i * 16, 16)] = k
                    tvb[pl.ds(i * 16, 16)] = gbase + lanes
                    posb[i // 8, pl.ds((i % 8) * 16, 16)] = pos
                    return 0
                lax.fori_loop(0, vpw, p_v, 0)

                _scatter_chunks(posb, sem, nchunk,
                                [(tkb, keys2), (tvb, vals2)])
                return 0
            lax.fori_loop(0, nwin, p_win, 0)
            return 0

        lax.fori_loop(0, rpw, row_loop, 0)

    return functools.partial(
        pl.kernel,
        mesh=plsc.VectorSubcoreMesh(**_MESH),
        out_type=(
            jax.ShapeDtypeStruct((rows * npad,), jnp.uint32),
            jax.ShapeDtypeStruct((rows * npad,), jnp.int32),
        ),
        scratch_types=[
            pltpu.VMEM((w,), jnp.float32),
            pltpu.VMEM((nchunk, 128), jnp.int32),
            pltpu.VMEM((w,), jnp.uint32),
            pltpu.VMEM((w,), jnp.int32),
            pltpu.VMEM((NBINS,), jnp.int32),
            pltpu.SemaphoreType.DMA,
        ],
        compiler_params=pltpu.CompilerParams(needs_layout_passes=False),
    )(body)


def _build_pass2(rows, n, w):
    npad = -(-n // w) * w
    nwin = npad // w
    vpw = w // 16
    nfull = n // w
    tail = n - nfull * w
    nchunk = w // 128
    rpw = rows // NWORK

    def body(keys2, vals2, finalpad, win_k, win_v, posb, hist, sem):
        wid = lax.axis_index("s") * 2 + lax.axis_index("c")

        ones = jnp.ones((16,), jnp.int32)

        def row_loop(j, _):
            r = wid * rpw + j
            rbase = r * npad

            _zero_hist(hist)

            def h_win(w_, _):
                pltpu.sync_copy(keys2.at[pl.ds(rbase + w_ * w, w)], win_k)

                def h_v(i, _):
                    k = win_k[pl.ds(i * 16, 16)]
                    d = (k >> jnp.uint32(16)).astype(jnp.int32)
                    plsc.addupdate_scatter(hist, [d], ones)
                    return 0
                lax.fori_loop(0, vpw, h_v, 0)
                return 0
            lax.fori_loop(0, nwin, h_win, 0)

            _scan_hist(hist, rbase)

            def p_win(w_, _):
                pltpu.sync_copy(keys2.at[pl.ds(rbase + w_ * w, w)], win_k)
                pltpu.sync_copy(vals2.at[pl.ds(rbase + w_ * w, w)], win_v)

                def p_v(i, _):
                    k = win_k[pl.ds(i * 16, 16)]
                    d = (k >> jnp.uint32(16)).astype(jnp.int32)
                    cnt, _last = plsc.scan_count(d)
                    base = plsc.load_gather(hist, [d])
                    pos = base + cnt - 1
                    plsc.addupdate_scatter(hist, [d], ones)
                    posb[i // 8, pl.ds((i % 8) * 16, 16)] = pos
                    return 0
                lax.fori_loop(0, vpw, p_v, 0)

                _scatter_chunks(posb, sem, nchunk, [(win_v, finalpad)])
                return 0
            lax.fori_loop(0, nwin, p_win, 0)
            return 0

        lax.fori_loop(0, rpw, row_loop, 0)

    return functools.partial(
        pl.kernel,
        mesh=plsc.VectorSubcoreMesh(**_MESH),
        out_type=jax.ShapeDtypeStruct((rows * npad,), jnp.int32),
        scratch_types=[
            pltpu.VMEM((w,), jnp.uint32),
            pltpu.VMEM((w,), jnp.int32),
            pltpu.VMEM((nchunk, 128), jnp.int32),
            pltpu.VMEM((NBINS,), jnp.int32),
            pltpu.SemaphoreType.DMA,
        ],
        compiler_params=pltpu.CompilerParams(needs_layout_passes=False),
    )(body)


def _run(inputs, w=2048):
    rows, n = inputs.shape
    npad = -(-n // w) * w
    keys2, vals2 = _build_pass1(rows, n, w)(inputs.reshape(-1))
    finalpad = _build_pass2(rows, n, w)(keys2, vals2)
    return finalpad.reshape(rows, npad)[:, :n]


@jax.jit
def kernel(inputs):
    return _run(inputs)


# speedup vs baseline: 1.0001x; 1.0001x over previous
import functools

import jax
import jax.numpy as jnp
import numpy as np
from jax import lax
from jax.experimental import pallas as pl
from jax.experimental.pallas import tpu as pltpu
from jax.experimental.pallas import tpu_sc as plsc

NWORK = 32
NBINS = 65536
SENTINEL = np.uint32(0xFFFFFFFF)
_MESH = dict(core_axis_name="c", subcore_axis_name="s", num_cores=2)


def _keys_from_f32(x):
    u = plsc.bitcast(x, jnp.uint32)
    u = jnp.where(u == jnp.uint32(0x80000000), jnp.uint32(0), u)
    neg = u >= jnp.uint32(0x80000000)
    return jnp.where(neg, u, (u ^ SENTINEL) & jnp.uint32(0x7FFFFFFF))


def _zero_hist(hist):
    def z(i, _):
        hist[pl.ds(i * 16, 16)] = jnp.zeros((16,), jnp.int32)
        return 0
    lax.fori_loop(0, NBINS // 16, z, 0)


def _scan_hist(hist, bias):
    def s(i, c):
        v = hist[pl.ds(i * 16, 16)]
        hist[pl.ds(i * 16, 16)] = c + plsc.cumsum(v) - v
        return c + jnp.sum(v)
    lax.fori_loop(0, NBINS // 16, s, bias)


def _scatter_chunks(posb, sem, nchunk, pairs):
    cps = [
        pltpu.async_copy(
            src_ref.at[pl.ds(c * 128, 128)],
            dst_ref.at[posb.at[c]],
            sem,
        )
        for src_ref, dst_ref in pairs
        for c in range(nchunk)
    ]
    for cp in cps:
        cp.wait()


def _build_pass1(rows, n, w):
    npad = -(-n // w) * w
    nwin = npad // w
    vpw = w // 16
    nfull = n // w
    tail = n - nfull * w
    nchunk = w // 128
    rpw = rows // NWORK
    assert rows % NWORK == 0 and w % 128 == 0 and tail % 8 == 0

    def body(x_hbm, keys2, vals2, win_f, posb, tkb, tvb, hist, sem):
        wid = lax.axis_index("s") * 2 + lax.axis_index("c")
        lanes = lax.iota(jnp.int32, 16)
        ones = jnp.ones((16,), jnp.int32)

        def row_loop(j, _):
            r = wid * rpw + j
            rbase = r * npad

            def load_x(w_):
                @pl.when(w_ < nfull)
                def _():
                    pltpu.sync_copy(x_hbm.at[pl.ds(r * n + w_ * w, w)], win_f)
                if tail:
                    @pl.when(w_ == nfull)
                    def _():
                        pltpu.sync_copy(
                            x_hbm.at[pl.ds(r * n + nfull * w, tail)],
                            win_f.at[pl.ds(0, tail)])

            _zero_hist(hist)

            def h_win(w_, _):
                load_x(w_)

                def h_v(i, _):
                    gbase = w_ * w + i * 16
                    valid = (gbase + lanes) < n
                    k = _keys_from_f32(win_f[pl.ds(i * 16, 16)])
                    k = jnp.where(valid, k, SENTINEL)
                    d = (k & jnp.uint32(0xFFFF)).astype(jnp.int32)
                    plsc.addupdate_scatter(hist, [d], ones)
                    return 0
                lax.fori_loop(0, vpw, h_v, 0)
                return 0
            lax.fori_loop(0, nwin, h_win, 0)

            _scan_hist(hist, rbase)

            def p_win(w_, _):
                load_x(w_)

                def p_v(i, _):
                    gbase = w_ * w + i * 16
                    valid = (gbase + lanes) < n
                    k = _keys_from_f32(win_f[pl.ds(i * 16, 16)])
                    k = jnp.where(valid, k, SENTINEL)
                    d = (k & jnp.uint32(0xFFFF)).astype(jnp.int32)
                    cnt, _last = plsc.scan_count(d)
                    base = plsc.load_gather(hist, [d])
                    pos = base + cnt - 1
                    plsc.addupdate_scatter(hist, [d], ones)
                    tkb[pl.ds(i * 16, 16)] = k
                    tvb[pl.ds(i * 16, 16)] = gbase + lanes
                    posb[i // 8, pl.ds((i % 8) * 16, 16)] = pos
                    return 0
                lax.fori_loop(0, vpw, p_v, 0)

                _scatter_chunks(posb, sem, nchunk,
                                [(tkb, keys2), (tvb, vals2)])
                return 0
            lax.fori_loop(0, nwin, p_win, 0)
            return 0

        lax.fori_loop(0, rpw, row_loop, 0)

    return functools.partial(
        pl.kernel,
        mesh=plsc.VectorSubcoreMesh(**_MESH),
        out_type=(
            jax.ShapeDtypeStruct((rows * npad,), jnp.uint32),
            jax.ShapeDtypeStruct((rows * npad,), jnp.int32),
        ),
        scratch_types=[
            pltpu.VMEM((w,), jnp.float32),
            pltpu.VMEM((nchunk, 128), jnp.int32),
            pltpu.VMEM((w,), jnp.uint32),
            pltpu.VMEM((w,), jnp.int32),
            pltpu.VMEM((NBINS,), jnp.int32),
            pltpu.SemaphoreType.DMA,
        ],
        compiler_params=pltpu.CompilerParams(needs_layout_passes=False),
    )(body)


def _build_pass2(rows, n, w):
    npad = -(-n // w) * w
    nwin = npad // w
    vpw = w // 16
    nfull = n // w
    tail = n - nfull * w
    nchunk = w // 128
    rpw = rows // NWORK

    def body(keys2, vals2, finalpad, win_k, win_v, posb, hist, sem):
        wid = lax.axis_index("s") * 2 + lax.axis_index("c")

        ones = jnp.ones((16,), jnp.int32)

        def row_loop(j, _):
            r = wid * rpw + j
            rbase = r * npad

            _zero_hist(hist)

            def h_win(w_, _):
                pltpu.sync_copy(keys2.at[pl.ds(rbase + w_ * w, w)], win_k)

                def h_v(i, _):
                    k = win_k[pl.ds(i * 16, 16)]
                    d = (k >> jnp.uint32(16)).astype(jnp.int32)
                    plsc.addupdate_scatter(hist, [d], ones)
                    return 0
                lax.fori_loop(0, vpw, h_v, 0)
                return 0
            lax.fori_loop(0, nwin, h_win, 0)

            _scan_hist(hist, rbase)

            def p_win(w_, _):
                pltpu.sync_copy(keys2.at[pl.ds(rbase + w_ * w, w)], win_k)
                pltpu.sync_copy(vals2.at[pl.ds(rbase + w_ * w, w)], win_v)

                def p_v(i, _):
                    k = win_k[pl.ds(i * 16, 16)]
                    d = (k >> jnp.uint32(16)).astype(jnp.int32)
                    cnt, _last = plsc.scan_count(d)
                    base = plsc.load_gather(hist, [d])
                    pos = base + cnt - 1
                    plsc.addupdate_scatter(hist, [d], ones)
                    posb[i // 8, pl.ds((i % 8) * 16, 16)] = pos
                    return 0
                lax.fori_loop(0, vpw, p_v, 0)

                _scatter_chunks(posb, sem, nchunk, [(win_v, finalpad)])
                return 0
            lax.fori_loop(0, nwin, p_win, 0)
            return 0

        lax.fori_loop(0, rpw, row_loop, 0)

    return functools.partial(
        pl.kernel,
        mesh=plsc.VectorSubcoreMesh(**_MESH),
        out_type=jax.ShapeDtypeStruct((rows * npad,), jnp.int32),
        scratch_types=[
            pltpu.VMEM((w,), jnp.uint32),
            pltpu.VMEM((w,), jnp.int32),
            pltpu.VMEM((nchunk, 128), jnp.int32),
            pltpu.VMEM((NBINS,), jnp.int32),
            pltpu.SemaphoreType.DMA,
        ],
        compiler_params=pltpu.CompilerParams(needs_layout_passes=False),
    )(body)


def _run(inputs, w=2048):
    rows, n = inputs.shape
    npad = -(-n // w) * w
    keys2, vals2 = _build_pass1(rows, n, w)(inputs.reshape(-1))
    finalpad = _build_pass2(rows, n, w)(keys2, vals2)
    return finalpad.reshape(rows, npad)[:, :n]


@jax.jit
def kernel(inputs):
    return _run(inputs)


# speedup vs baseline: 1.0027x; 1.0026x over previous
import functools

import jax
import jax.numpy as jnp
import numpy as np
from jax import lax
from jax.experimental import pallas as pl
from jax.experimental.pallas import tpu as pltpu
from jax.experimental.pallas import tpu_sc as plsc

NWORK = 32
NBINS = 65536
SENTINEL = np.uint32(0xFFFFFFFF)
_MESH = dict(core_axis_name="c", subcore_axis_name="s", num_cores=2)


def _keys_from_f32(x):
    u = plsc.bitcast(x, jnp.uint32)
    u = jnp.where(u == jnp.uint32(0x80000000), jnp.uint32(0), u)
    neg = u >= jnp.uint32(0x80000000)
    return jnp.where(neg, u, (u ^ SENTINEL) & jnp.uint32(0x7FFFFFFF))


def _zero_hist(hist):
    def z(i, _):
        hist[pl.ds(i * 16, 16)] = jnp.zeros((16,), jnp.int32)
        return 0
    lax.fori_loop(0, NBINS // 16, z, 0)


def _scan_hist(hist, bias):
    def s(i, c):
        v = hist[pl.ds(i * 16, 16)]
        hist[pl.ds(i * 16, 16)] = c + plsc.cumsum(v) - v
        return c + jnp.sum(v)
    lax.fori_loop(0, NBINS // 16, s, bias)


def _scatter_chunks(posb, sem, nchunk, pairs):
    cps = [
        pltpu.async_copy(
            src_ref.at[pl.ds(c * 128, 128)],
            dst_ref.at[posb.at[c]],
            sem,
        )
        for src_ref, dst_ref in pairs
        for c in range(nchunk)
    ]
    for cp in cps:
        cp.wait()


def _build_pass1(rows, n, w):
    npad = -(-n // w) * w
    nwin = npad // w
    vpw = w // 16
    nfull = n // w
    tail = n - nfull * w
    nchunk = w // 128
    rpw = rows // NWORK
    assert rows % NWORK == 0 and w % 128 == 0 and tail % 8 == 0

    def body(x_hbm, keys2, vals2, win_f, posb, tkb, tvb, hist, sem):
        wid = lax.axis_index("s") * 2 + lax.axis_index("c")
        lanes = lax.iota(jnp.int32, 16)
        ones = jnp.ones((16,), jnp.int32)

        def row_loop(j, _):
            r = wid * rpw + j
            rbase = r * npad

            def load_x(w_):
                @pl.when(w_ < nfull)
                def _():
                    pltpu.sync_copy(x_hbm.at[pl.ds(r * n + w_ * w, w)], win_f)
                if tail:
                    @pl.when(w_ == nfull)
                    def _():
                        pltpu.sync_copy(
                            x_hbm.at[pl.ds(r * n + nfull * w, tail)],
                            win_f.at[pl.ds(0, tail)])

            _zero_hist(hist)

            def h_win(w_, _):
                load_x(w_)

                @plsc.parallel_loop(0, vpw, unroll=8)
                def h_v(i):
                    gbase = w_ * w + i * 16
                    valid = (gbase + lanes) < n
                    k = _keys_from_f32(win_f[pl.ds(i * 16, 16)])
                    k = jnp.where(valid, k, SENTINEL)
                    d = (k & jnp.uint32(0xFFFF)).astype(jnp.int32)
                    plsc.addupdate_scatter(hist, [d], ones)
                return 0
            lax.fori_loop(0, nwin, h_win, 0)

            _scan_hist(hist, rbase)

            def p_win(w_, _):
                load_x(w_)

                def p_v(i, _):
                    gbase = w_ * w + i * 16
                    valid = (gbase + lanes) < n
                    k = _keys_from_f32(win_f[pl.ds(i * 16, 16)])
                    k = jnp.where(valid, k, SENTINEL)
                    d = (k & jnp.uint32(0xFFFF)).astype(jnp.int32)
                    cnt, _last = plsc.scan_count(d)
                    base = plsc.load_gather(hist, [d])
                    pos = base + cnt - 1
                    plsc.addupdate_scatter(hist, [d], ones)
                    tkb[pl.ds(i * 16, 16)] = k
                    tvb[pl.ds(i * 16, 16)] = gbase + lanes
                    posb[i // 8, pl.ds((i % 8) * 16, 16)] = pos
                    return 0
                lax.fori_loop(0, vpw, p_v, 0)

                _scatter_chunks(posb, sem, nchunk,
                                [(tkb, keys2), (tvb, vals2)])
                return 0
            lax.fori_loop(0, nwin, p_win, 0)
            return 0

        lax.fori_loop(0, rpw, row_loop, 0)

    return functools.partial(
        pl.kernel,
        mesh=plsc.VectorSubcoreMesh(**_MESH),
        out_type=(
            jax.ShapeDtypeStruct((rows * npad,), jnp.uint32),
            jax.ShapeDtypeStruct((rows * npad,), jnp.int32),
        ),
        scratch_types=[
            pltpu.VMEM((w,), jnp.float32),
            pltpu.VMEM((nchunk, 128), jnp.int32),
            pltpu.VMEM((w,), jnp.uint32),
            pltpu.VMEM((w,), jnp.int32),
            pltpu.VMEM((NBINS,), jnp.int32),
            pltpu.SemaphoreType.DMA,
        ],
        compiler_params=pltpu.CompilerParams(needs_layout_passes=False),
    )(body)


def _build_pass2(rows, n, w):
    npad = -(-n // w) * w
    nwin = npad // w
    vpw = w // 16
    nfull = n // w
    tail = n - nfull * w
    nchunk = w // 128
    rpw = rows // NWORK

    def body(keys2, vals2, finalpad, win_k, win_v, posb, hist, sem):
        wid = lax.axis_index("s") * 2 + lax.axis_index("c")

        ones = jnp.ones((16,), jnp.int32)

        def row_loop(j, _):
            r = wid * rpw + j
            rbase = r * npad

            _zero_hist(hist)

            def h_win(w_, _):
                pltpu.sync_copy(keys2.at[pl.ds(rbase + w_ * w, w)], win_k)

                @plsc.parallel_loop(0, vpw, unroll=8)
                def h_v(i):
                    k = win_k[pl.ds(i * 16, 16)]
                    d = (k >> jnp.uint32(16)).astype(jnp.int32)
                    plsc.addupdate_scatter(hist, [d], ones)
                return 0
            lax.fori_loop(0, nwin, h_win, 0)

            _scan_hist(hist, rbase)

            def p_win(w_, _):
                pltpu.sync_copy(keys2.at[pl.ds(rbase + w_ * w, w)], win_k)
                pltpu.sync_copy(vals2.at[pl.ds(rbase + w_ * w, w)], win_v)

                def p_v(i, _):
                    k = win_k[pl.ds(i * 16, 16)]
                    d = (k >> jnp.uint32(16)).astype(jnp.int32)
                    cnt, _last = plsc.scan_count(d)
                    base = plsc.load_gather(hist, [d])
                    pos = base + cnt - 1
                    plsc.addupdate_scatter(hist, [d], ones)
                    posb[i // 8, pl.ds((i % 8) * 16, 16)] = pos
                    return 0
                lax.fori_loop(0, vpw, p_v, 0)

                _scatter_chunks(posb, sem, nchunk, [(win_v, finalpad)])
                return 0
            lax.fori_loop(0, nwin, p_win, 0)
            return 0

        lax.fori_loop(0, rpw, row_loop, 0)

    return functools.partial(
        pl.kernel,
        mesh=plsc.VectorSubcoreMesh(**_MESH),
        out_type=jax.ShapeDtypeStruct((rows * npad,), jnp.int32),
        scratch_types=[
            pltpu.VMEM((w,), jnp.uint32),
            pltpu.VMEM((w,), jnp.int32),
            pltpu.VMEM((nchunk, 128), jnp.int32),
            pltpu.VMEM((NBINS,), jnp.int32),
            pltpu.SemaphoreType.DMA,
        ],
        compiler_params=pltpu.CompilerParams(needs_layout_passes=False),
    )(body)


def _run(inputs, w=2048):
    rows, n = inputs.shape
    npad = -(-n // w) * w
    keys2, vals2 = _build_pass1(rows, n, w)(inputs.reshape(-1))
    finalpad = _build_pass2(rows, n, w)(keys2, vals2)
    return finalpad.reshape(rows, npad)[:, :n]


@jax.jit
def kernel(inputs):
    return _run(inputs)
